# baseline (speedup 1.0000x reference)
import functools

import jax
import jax.numpy as jnp
from jax import lax
from jax.experimental import pallas as pl
from jax.experimental.pallas import tpu as pltpu
from jax.experimental.pallas import tpu_sc as plsc

BATCH = 16384
EMBED_DIM = 32
L = 16

_NC = 2
_NS = 16
NW = _NC * _NS
B_PER_W = BATCH // NW


def _body(ex_hbm, tgt_hbm, ctx_hbm, out_hbm,
          ex_v, idx_t_v, idx_c_v, rows_t_v, rows_c_v, out_v,
          sem_t, sem_c):
    wid = lax.axis_index("s") * _NC + lax.axis_index("c")
    base = wid * B_PER_W

    lanes = lax.iota(jnp.int32, L)
    zeros = jnp.zeros((L,), jnp.int32)
    ones = jnp.ones((L,), jnp.int32)

    pltpu.sync_copy(ex_hbm.at[pl.ds(base, B_PER_W), :], ex_v)

    def extract(k, _):
        rows = lanes + k * L
        t = plsc.load_gather(ex_v, [rows, zeros])
        c = plsc.load_gather(ex_v, [rows, ones])
        idx_t_v[pl.ds(k * L, L)] = t
        idx_c_v[pl.ds(k * L, L)] = c
        return _

    lax.fori_loop(0, B_PER_W // L, extract, None)

    cp_t = pltpu.async_copy(tgt_hbm.at[idx_t_v], rows_t_v, sem_t)
    cp_c = pltpu.async_copy(ctx_hbm.at[idx_c_v], rows_c_v, sem_c)
    cp_t.wait()
    cp_c.wait()

    def compute(k, _):
        rows = lanes + k * L
        acc = jnp.zeros((L,), jnp.float32)
        for d in range(EMBED_DIM):
            dcol = jnp.full((L,), d, jnp.int32)
            t = plsc.load_gather(rows_t_v, [rows, dcol])
            c = plsc.load_gather(rows_c_v, [rows, dcol])
            acc = acc + t * c
        out_v[pl.ds(k * L, L)] = 1.0 / (1.0 + jnp.exp(-acc))
        return _

    lax.fori_loop(0, B_PER_W // L, compute, None)

    pltpu.sync_copy(out_v, out_hbm.at[pl.ds(base, B_PER_W)])


def kernel(examples, target_embeddings, context_embeddings):
    mesh = plsc.VectorSubcoreMesh(core_axis_name="c", subcore_axis_name="s")
    k = functools.partial(
        pl.kernel,
        mesh=mesh,
        compiler_params=pltpu.CompilerParams(
            needs_layout_passes=False,
            use_tc_tiling_on_sc=False,
        ),
        out_type=jax.ShapeDtypeStruct((BATCH,), jnp.float32),
        scratch_types=[
            pltpu.VMEM((B_PER_W, 3), jnp.int32),
            pltpu.VMEM((B_PER_W,), jnp.int32),
            pltpu.VMEM((B_PER_W,), jnp.int32),
            pltpu.VMEM((B_PER_W, EMBED_DIM), jnp.float32),
            pltpu.VMEM((B_PER_W, EMBED_DIM), jnp.float32),
            pltpu.VMEM((B_PER_W,), jnp.float32),
            pltpu.SemaphoreType.DMA,
            pltpu.SemaphoreType.DMA,
        ],
    )(_body)
    return k(examples, target_embeddings, context_embeddings)


# speedup vs baseline: 1.0018x; 1.0018x over previous
import functools

import jax
import jax.numpy as jnp
from jax import lax
from jax.experimental import pallas as pl
from jax.experimental.pallas import tpu as pltpu
from jax.experimental.pallas import tpu_sc as plsc

VOCAB = 1000000
BATCH = 16384
EMBED_DIM = 32
L = 16
PACK = 128 // EMBED_DIM

_NC = 2
_NS = 16
NW = _NC * _NS
B_PER_W = BATCH // NW
CHUNK = 256
N_CHUNKS = B_PER_W // CHUNK


def _body(ex_hbm, tgt_hbm, ctx_hbm, out_hbm,
          ex_v, idx_t_v, idx_c_v, off_t_v, off_c_v,
          rows_t_v, rows_c_v, out_v, sem_t, sem_c):
    wid = lax.axis_index("s") * _NC + lax.axis_index("c")
    base = wid * B_PER_W

    pltpu.sync_copy(ex_hbm.at[pl.ds(base * 3, B_PER_W * 3)], ex_v)

    lanes = lax.iota(jnp.int32, L)

    for chunk in range(N_CHUNKS):
        cbase = chunk * CHUNK

        def extract(k, _):
            flat = (lanes + cbase + k * L) * 3
            t = plsc.load_gather(ex_v, [flat])
            c = plsc.load_gather(ex_v, [flat + 1])
            idx_t_v[pl.ds(k * L, L)] = t >> 2
            idx_c_v[pl.ds(k * L, L)] = c >> 2
            off_t_v[pl.ds(k * L, L)] = (t & 3) * EMBED_DIM
            off_c_v[pl.ds(k * L, L)] = (c & 3) * EMBED_DIM
            return _

        lax.fori_loop(0, CHUNK // L, extract, None)

        cp_t = pltpu.async_copy(tgt_hbm.at[idx_t_v], rows_t_v, sem_t)
        cp_c = pltpu.async_copy(ctx_hbm.at[idx_c_v], rows_c_v, sem_c)
        cp_t.wait()
        cp_c.wait()

        def compute(k, _):
            rows = lanes + k * L
            off_t = off_t_v[pl.ds(k * L, L)]
            off_c = off_c_v[pl.ds(k * L, L)]
            acc = jnp.zeros((L,), jnp.float32)
            for d in range(EMBED_DIM):
                t = plsc.load_gather(rows_t_v, [rows, off_t + d])
                c = plsc.load_gather(rows_c_v, [rows, off_c + d])
                acc = acc + t * c
            out_v[pl.ds(cbase + k * L, L)] = 1.0 / (1.0 + jnp.exp(-acc))
            return _

        lax.fori_loop(0, CHUNK // L, compute, None)

    pltpu.sync_copy(out_v, out_hbm.at[pl.ds(base, B_PER_W)])


def kernel(examples, target_embeddings, context_embeddings):
    mesh = plsc.VectorSubcoreMesh(core_axis_name="c", subcore_axis_name="s")
    k = functools.partial(
        pl.kernel,
        mesh=mesh,
        compiler_params=pltpu.CompilerParams(
            needs_layout_passes=False,
        ),
        out_type=jax.ShapeDtypeStruct((BATCH,), jnp.float32),
        scratch_types=[
            pltpu.VMEM((B_PER_W * 3,), jnp.int32),
            pltpu.VMEM((CHUNK,), jnp.int32),
            pltpu.VMEM((CHUNK,), jnp.int32),
            pltpu.VMEM((CHUNK,), jnp.int32),
            pltpu.VMEM((CHUNK,), jnp.int32),
            pltpu.VMEM((CHUNK, 128), jnp.float32),
            pltpu.VMEM((CHUNK, 128), jnp.float32),
            pltpu.VMEM((B_PER_W,), jnp.float32),
            pltpu.SemaphoreType.DMA,
            pltpu.SemaphoreType.DMA,
        ],
    )(_body)
    return k(
        examples.reshape(-1),
        target_embeddings.reshape(VOCAB // PACK, 128),
        context_embeddings.reshape(VOCAB // PACK, 128),
    )
